# baseline (speedup 1.0000x reference)
import functools

import jax
import jax.numpy as jnp
from jax import lax
from jax.experimental import pallas as pl
from jax.experimental.pallas import tpu as pltpu
from jax.experimental.pallas import tpu_sc as plsc

NUM_WORKERS = 32
CHUNK = 40
NBUF = 2
DPAD = 1024


def _gather_rows(idx, table_padded, d):
    n, = idx.shape
    per_w = n // NUM_WORKERS
    n_chunks = per_w // CHUNK

    mesh = plsc.VectorSubcoreMesh(core_axis_name="c", subcore_axis_name="s")

    @functools.partial(
        pl.kernel,
        mesh=mesh,
        out_type=jax.ShapeDtypeStruct((n, d), jnp.float32),
        scratch_types=[
            pltpu.VMEM((per_w,), jnp.int32),
            pltpu.VMEM((CHUNK, DPAD), jnp.float32),
            pltpu.VMEM((CHUNK, DPAD), jnp.float32),
            pltpu.SemaphoreType.DMA,
            pltpu.SemaphoreType.DMA,
            pltpu.SemaphoreType.DMA,
            pltpu.SemaphoreType.DMA,
        ],
        compiler_params=pltpu.CompilerParams(use_tc_tiling_on_sc=False),
    )
    def k(idx_hbm, table_hbm, out_hbm, idx_v, buf0, buf1, g0, g1, s0, s1):
        bufs = (buf0, buf1)
        g_sems = (g0, g1)
        s_sems = (s0, s1)
        wid = lax.axis_index("s") * 2 + lax.axis_index("c")
        base = wid * per_w
        pltpu.sync_copy(idx_hbm.at[pl.ds(base, per_w)], idx_v)

        def pair_body(p, carry):
            for b in range(NBUF):
                j = NBUF * p + b
                off = j * CHUNK

                @pl.when(p > 0)
                def _():
                    pltpu.make_async_copy(
                        bufs[b].at[:, pl.ds(0, d)],
                        out_hbm.at[pl.ds(base + off, CHUNK)],
                        s_sems[b],
                    ).wait()

                pltpu.async_copy(
                    table_hbm.at[idx_v.at[pl.ds(off, CHUNK)]], bufs[b], g_sems[b]
                ).wait()
                pltpu.async_copy(
                    bufs[b].at[:, pl.ds(0, d)],
                    out_hbm.at[pl.ds(base + off, CHUNK)],
                    s_sems[b],
                )
            return carry

        lax.fori_loop(0, n_chunks // NBUF, pair_body, 0)

        for b in range(NBUF):
            j = n_chunks - NBUF + b
            pltpu.make_async_copy(
                bufs[b].at[:, pl.ds(0, d)],
                out_hbm.at[pl.ds(base + j * CHUNK, CHUNK)],
                s_sems[b],
            ).wait()

    return k(idx, table_padded)


def kernel(context, table):
    b, s = context.shape
    v, d = table.shape
    idx = context.reshape(b * s).astype(jnp.int32)
    table_padded = jnp.pad(table, ((0, 0), (0, DPAD - d)))
    out = _gather_rows(idx, table_padded, d)
    return out.reshape(b, s, d)


# speedup vs baseline: 2.1782x; 2.1782x over previous
import functools

import jax
import jax.numpy as jnp
from jax import lax
from jax.experimental import pallas as pl
from jax.experimental.pallas import tpu as pltpu
from jax.experimental.pallas import tpu_sc as plsc

NUM_WORKERS = 32
CHUNK = 40
NBUF = 2
DPAD = 1024


def _gather_rows(idx, table_padded):
    n, = idx.shape
    per_w = n // NUM_WORKERS
    n_chunks = per_w // CHUNK

    mesh = plsc.VectorSubcoreMesh(core_axis_name="c", subcore_axis_name="s")

    @functools.partial(
        pl.kernel,
        mesh=mesh,
        out_type=jax.ShapeDtypeStruct((n, DPAD), jnp.float32),
        scratch_types=[
            pltpu.VMEM((per_w,), jnp.int32),
            pltpu.VMEM((CHUNK, DPAD), jnp.float32),
            pltpu.VMEM((CHUNK, DPAD), jnp.float32),
            pltpu.SemaphoreType.DMA,
            pltpu.SemaphoreType.DMA,
            pltpu.SemaphoreType.DMA,
            pltpu.SemaphoreType.DMA,
        ],
    )
    def k(idx_hbm, table_hbm, out_hbm, idx_v, buf0, buf1, g0, g1, s0, s1):
        bufs = (buf0, buf1)
        g_sems = (g0, g1)
        s_sems = (s0, s1)
        wid = lax.axis_index("s") * 2 + lax.axis_index("c")
        base = wid * per_w
        pltpu.sync_copy(idx_hbm.at[pl.ds(base, per_w)], idx_v)

        def pair_body(p, carry):
            for b in range(NBUF):
                j = NBUF * p + b
                off = j * CHUNK

                @pl.when(p > 0)
                def _():
                    pltpu.make_async_copy(
                        bufs[b],
                        out_hbm.at[pl.ds(base + off, CHUNK)],
                        s_sems[b],
                    ).wait()

                pltpu.async_copy(
                    table_hbm.at[idx_v.at[pl.ds(off, CHUNK)]], bufs[b], g_sems[b]
                ).wait()
                pltpu.async_copy(
                    bufs[b], out_hbm.at[pl.ds(base + off, CHUNK)], s_sems[b]
                )
            return carry

        lax.fori_loop(0, n_chunks // NBUF, pair_body, 0)

        for b in range(NBUF):
            j = n_chunks - NBUF + b
            pltpu.make_async_copy(
                bufs[b], out_hbm.at[pl.ds(base + j * CHUNK, CHUNK)], s_sems[b]
            ).wait()

    return k(idx, table_padded)


def _transpose_planes(g3, d):
    s, bb, dp = g3.shape

    def body(x_ref, o_ref):
        o_ref[0] = jnp.transpose(x_ref[0])[:d, :]

    return pl.pallas_call(
        body,
        grid=(s,),
        in_specs=[pl.BlockSpec((1, bb, dp), lambda i: (i, 0, 0))],
        out_specs=pl.BlockSpec((1, d, bb), lambda i: (i, 0, 0)),
        out_shape=jax.ShapeDtypeStruct((s, d, bb), jnp.float32),
    )(g3)


def kernel(context, table):
    b, s = context.shape
    v, d = table.shape
    idx = context.T.reshape(b * s).astype(jnp.int32)
    table_padded = jnp.pad(table, ((0, 0), (0, DPAD - d)))
    g = _gather_rows(idx, table_padded)
    out5 = _transpose_planes(g.reshape(s, b, DPAD), d)
    return jnp.transpose(out5, (2, 0, 1))
